# Initial kernel scaffold; baseline (speedup 1.0000x reference)
import jax
import jax.numpy as jnp
from jax.experimental import pallas as pl


def kernel(x, edge_index, W1, b1, W2, b2, Wm1, bm1, Wm2, bm2):
    raise NotImplementedError("write your pallas kernel here")


# speedup vs baseline: 4.7854x; 4.7854x over previous
import functools
import math

import jax
import jax.numpy as jnp
import numpy as np
from numpy.polynomial.chebyshev import chebfit
from jax import lax
from jax.experimental import pallas as pl
from jax.experimental.pallas import tpu as pltpu
from jax.experimental.pallas import tpu_sc as plsc

_N = 10000
_D = 128
_NTRASH = 112
_NT = _N + _NTRASH
_NSC = 2
_NSUB = 16
_NW = _NSC * _NSUB
_RPT = _NT // _NSUB
_CH = 128
_BLK = 400


def _chi_thetas():
    thetas = []
    for d in (2, 4):
        n = 2 * d
        i = n / 2 - 1
        m = 1.0 / (i + 2)
        n = int(n)

        def y(t):
            return (1.0 / (2 ** (n / 2) * math.gamma(n / 2))
                    * (t / m) ** (n / 2 - 1) * np.exp(-(t / m) / 2))

        xs = np.linspace(0.0, 2.0, 8001)
        s = np.trapz(y(xs), xs)
        lambs = np.linspace(0, 2, 500)
        coef = chebfit(lambs, y(lambs) / s, 3)
        thetas.append([float(v) for v in coef[::-1]])
    return thetas


_T1, _T2 = _chi_thetas()

_MESH_KW = dict(core_axis_name="c", subcore_axis_name="s")


def _deg_kernel(dst_pad, zeros128, ones128):
    ep = dst_pad.shape[0]
    epw = ep // _NW
    nch = epw // _CH

    @functools.partial(
        pl.kernel,
        out_type=jax.ShapeDtypeStruct((_NSC, _NT, _D), jnp.float32),
        mesh=plsc.VectorSubcoreMesh(**_MESH_KW),
        scratch_types=[
            pltpu.VMEM((_CH,), jnp.int32),
            pltpu.VMEM((_CH, _D), jnp.float32),
            pltpu.VMEM_SHARED((_NT, _D), jnp.float32),
        ],
    )
    def k(dst_hbm, z_hbm, ones_hbm, out_hbm, didx, ones_v, acc):
        c = lax.axis_index("c")
        s = lax.axis_index("s")
        row0 = s * _RPT
        pltpu.sync_copy(z_hbm.at[pl.ds(row0, _RPT)], acc.at[pl.ds(row0, _RPT)])
        pltpu.sync_copy(ones_hbm, ones_v)
        plsc.subcore_barrier()
        base = (s * _NSC + c) * epw

        @pl.loop(0, nch)
        def _(j):
            pltpu.sync_copy(dst_hbm.at[pl.ds(base + j * _CH, _CH)], didx)
            pltpu.sync_copy(ones_v, acc.at[didx], add=True)

        plsc.subcore_barrier()
        pltpu.sync_copy(acc.at[pl.ds(row0, _RPT)],
                        out_hbm.at[c, pl.ds(row0, _RPT)])

    return k(dst_pad, zeros128, ones128)


def _agg_kernel(g, src_pad, dst_pad, zeros128):
    ep = src_pad.shape[0]
    epw = ep // _NW
    nch = epw // _CH

    @functools.partial(
        pl.kernel,
        out_type=jax.ShapeDtypeStruct((_NSC, _NT, _D), jnp.float32),
        mesh=plsc.VectorSubcoreMesh(**_MESH_KW),
        scratch_types=[
            pltpu.VMEM((_CH,), jnp.int32),
            pltpu.VMEM((_CH,), jnp.int32),
            pltpu.VMEM((_CH, _D), jnp.float32),
            pltpu.VMEM_SHARED((_NT, _D), jnp.float32),
        ],
    )
    def k(g_hbm, src_hbm, dst_hbm, z_hbm, out_hbm, sidx, didx, rows, acc):
        c = lax.axis_index("c")
        s = lax.axis_index("s")
        row0 = s * _RPT
        pltpu.sync_copy(z_hbm.at[pl.ds(row0, _RPT)], acc.at[pl.ds(row0, _RPT)])
        plsc.subcore_barrier()
        base = (s * _NSC + c) * epw

        @pl.loop(0, nch)
        def _(j):
            off = base + j * _CH
            pltpu.sync_copy(src_hbm.at[pl.ds(off, _CH)], sidx)
            pltpu.sync_copy(dst_hbm.at[pl.ds(off, _CH)], didx)
            pltpu.sync_copy(g_hbm.at[sidx], rows)
            pltpu.sync_copy(rows, acc.at[didx], add=True)

        plsc.subcore_barrier()
        pltpu.sync_copy(acc.at[pl.ds(row0, _RPT)],
                        out_hbm.at[c, pl.ds(row0, _RPT)])

    return k(g, src_pad, dst_pad, zeros128)


def _mlp_body(x_ref, w1_ref, b1_ref, w2_ref, b2_ref, degp_ref, g0_ref, dc_ref):
    h = jnp.dot(x_ref[...], w1_ref[...], preferred_element_type=jnp.float32)
    h = jnp.maximum(h + b1_ref[...], 0.0)
    h = jnp.dot(h, w2_ref[...], preferred_element_type=jnp.float32)
    h = jnp.maximum(h + b2_ref[...], 0.0)
    deg = degp_ref[0, :, 0:1] + degp_ref[1, :, 0:1]
    dc = jnp.maximum(deg, 1.0)
    dc_ref[...] = dc
    g0_ref[...] = h * lax.rsqrt(dc)


def _mlp_call(x, W1, b1, W2, b2, degp):
    nb = _N // _BLK
    return pl.pallas_call(
        _mlp_body,
        grid=(nb,),
        in_specs=[
            pl.BlockSpec((_BLK, _D), lambda i: (i, 0)),
            pl.BlockSpec((_D, _D), lambda i: (0, 0)),
            pl.BlockSpec((1, _D), lambda i: (0, 0)),
            pl.BlockSpec((_D, _D), lambda i: (0, 0)),
            pl.BlockSpec((1, _D), lambda i: (0, 0)),
            pl.BlockSpec((_NSC, _BLK, _D), lambda i: (0, i, 0)),
        ],
        out_specs=[
            pl.BlockSpec((_BLK, _D), lambda i: (i, 0)),
            pl.BlockSpec((_BLK, 1), lambda i: (i, 0)),
        ],
        out_shape=[
            jax.ShapeDtypeStruct((_N, _D), jnp.float32),
            jax.ShapeDtypeStruct((_N, 1), jnp.float32),
        ],
    )(x, W1, b1.reshape(1, _D), W2, b2.reshape(1, _D), degp)


def _comb_body(g_ref, ap_ref, dc_ref, o_ref):
    agg = ap_ref[0] + ap_ref[1]
    o_ref[...] = g_ref[...] - agg / dc_ref[...]


def _comb_call(g, aggp, dc):
    nb = _N // _BLK
    return pl.pallas_call(
        _comb_body,
        grid=(nb,),
        in_specs=[
            pl.BlockSpec((_BLK, _D), lambda i: (i, 0)),
            pl.BlockSpec((_NSC, _BLK, _D), lambda i: (0, i, 0)),
            pl.BlockSpec((_BLK, 1), lambda i: (i, 0)),
        ],
        out_specs=pl.BlockSpec((_BLK, _D), lambda i: (i, 0)),
        out_shape=jax.ShapeDtypeStruct((_N, _D), jnp.float32),
    )(g, aggp, dc)


def _final_body(g0_ref, gs_ref, dc_ref, wm1_ref, bm1_ref,
                wm2_ref, bm2_ref, o_ref):
    dc = dc_ref[...]
    sdeg = jnp.sqrt(dc)
    wm1 = wm1_ref[...]
    wa = wm1[0:_D]
    wb = wm1[_D:2 * _D]
    wc = wm1[2 * _D:3 * _D]
    w0 = wa + _T1[0] * wb + _T2[0] * wc
    u = jnp.dot(sdeg * g0_ref[...], w0, preferred_element_type=jnp.float32)
    u = u + jnp.dot(sdeg * gs_ref[0], _T1[1] * wb + _T2[1] * wc,
                    preferred_element_type=jnp.float32)
    u = u + jnp.dot(sdeg * gs_ref[1], _T1[2] * wb + _T2[2] * wc,
                    preferred_element_type=jnp.float32)
    u = u + jnp.dot(sdeg * gs_ref[2], _T1[3] * wb + _T2[3] * wc,
                    preferred_element_type=jnp.float32)
    r = jnp.maximum(u + bm1_ref[...], 0.0)
    o_ref[...] = jnp.dot(r, wm2_ref[...],
                         preferred_element_type=jnp.float32) + bm2_ref[...]


def _final_call(g0, gs, dc, Wm1, bm1, Wm2, bm2):
    nb = _N // _BLK
    nc = Wm2.shape[1]
    return pl.pallas_call(
        _final_body,
        grid=(nb,),
        in_specs=[
            pl.BlockSpec((_BLK, _D), lambda i: (i, 0)),
            pl.BlockSpec((3, _BLK, _D), lambda i: (0, i, 0)),
            pl.BlockSpec((_BLK, 1), lambda i: (i, 0)),
            pl.BlockSpec((3 * _D, _D), lambda i: (0, 0)),
            pl.BlockSpec((1, _D), lambda i: (0, 0)),
            pl.BlockSpec((_D, nc), lambda i: (0, 0)),
            pl.BlockSpec((1, nc), lambda i: (0, 0)),
        ],
        out_specs=pl.BlockSpec((_BLK, nc), lambda i: (i, 0)),
        out_shape=jax.ShapeDtypeStruct((_N, nc), jnp.float32),
    )(g0, gs, dc, Wm1, bm1.reshape(1, _D), Wm2,
      bm2.reshape(1, nc))


def kernel(x, edge_index, W1, b1, W2, b2, Wm1, bm1, Wm2, bm2):
    src = edge_index[0].astype(jnp.int32)
    dst = edge_index[1].astype(jnp.int32)
    e = src.shape[0]
    unit = _NW * _CH
    ep = ((e + unit - 1) // unit) * unit
    npad = ep - e
    pad_i = jnp.arange(npad, dtype=jnp.int32)
    src_pad = jnp.concatenate([src, (pad_i * 997) % _N])
    dst_pad = jnp.concatenate([dst, _N + (pad_i % _NTRASH)])

    zeros128 = jnp.zeros((_NT, _D), jnp.float32)
    ones128 = jnp.ones((_CH, _D), jnp.float32)

    degp = _deg_kernel(dst_pad, zeros128, ones128)
    g0, dc = _mlp_call(x, W1, b1, W2, b2, degp)

    def step(g, _):
        a = _agg_kernel(g, src_pad, dst_pad, zeros128)
        gn = _comb_call(g, a, dc)
        return gn, gn

    _, gs = lax.scan(step, g0, xs=None, length=3)
    return _final_call(g0, gs, dc, Wm1, bm1, Wm2, bm2)


# speedup vs baseline: 8.3291x; 1.7405x over previous
import functools
import math

import jax
import jax.numpy as jnp
import numpy as np
from numpy.polynomial.chebyshev import chebfit
from jax import lax
from jax.experimental import pallas as pl
from jax.experimental.pallas import tpu as pltpu
from jax.experimental.pallas import tpu_sc as plsc

_N = 10000
_D = 128
_NTRASH = 112
_NT = _N + _NTRASH
_NSC = 2
_NSUB = 16
_NW = _NSC * _NSUB
_RPT = _NT // _NSUB
_CH = 128
_BLK = 400


def _chi_thetas():
    thetas = []
    for d in (2, 4):
        n = 2 * d
        i = n / 2 - 1
        m = 1.0 / (i + 2)
        n = int(n)

        def y(t):
            return (1.0 / (2 ** (n / 2) * math.gamma(n / 2))
                    * (t / m) ** (n / 2 - 1) * np.exp(-(t / m) / 2))

        xs = np.linspace(0.0, 2.0, 8001)
        s = np.trapz(y(xs), xs)
        lambs = np.linspace(0, 2, 500)
        coef = chebfit(lambs, y(lambs) / s, 3)
        thetas.append([float(v) for v in coef[::-1]])
    return thetas


_T1, _T2 = _chi_thetas()

_MESH_KW = dict(core_axis_name="c", subcore_axis_name="s")


def _deg_kernel(dst_pad, zeros128, ones128, nch):
    epw = nch * _CH

    @functools.partial(
        pl.kernel,
        out_type=jax.ShapeDtypeStruct((_NSC, _NT, _D), jnp.float32),
        mesh=plsc.VectorSubcoreMesh(**_MESH_KW),
        scratch_types=[
            [pltpu.VMEM((_CH,), jnp.int32) for _ in range(4)],
            pltpu.VMEM((_CH, _D), jnp.float32),
            pltpu.VMEM_SHARED((_NT, _D), jnp.float32),
            [pltpu.SemaphoreType.DMA for _ in range(4)],
        ],
    )
    def k(dst_hbm, z_hbm, ones_hbm, out_hbm, didx, ones_v, acc, dsem):
        c = lax.axis_index("c")
        s = lax.axis_index("s")
        row0 = s * _RPT
        pltpu.sync_copy(z_hbm.at[pl.ds(row0, _RPT)], acc.at[pl.ds(row0, _RPT)])
        pltpu.sync_copy(ones_hbm, ones_v)
        plsc.subcore_barrier()
        base = (s * _NSC + c) * epw

        def start_idx(j, q):
            pltpu.async_copy(dst_hbm.at[pl.ds(base + j * _CH, _CH)],
                             didx[q], dsem[q])

        def wait_idx(j, q):
            pltpu.make_async_copy(dst_hbm.at[pl.ds(base + j * _CH, _CH)],
                                  didx[q], dsem[q]).wait()

        def scatter(q):
            pltpu.sync_copy(ones_v, acc.at[didx[q]], add=True)

        start_idx(0, 0)
        start_idx(1, 1)

        @pl.loop(0, (nch - 4) // 4)
        def _(jj):
            for u in range(4):
                j = jj * 4 + u
                q = u % 4
                wait_idx(j, q)
                start_idx(j + 2, (u + 2) % 4)
                scatter(q)

        for u in range(4):
            j = nch - 4 + u
            q = (nch - 4 + u) % 4
            wait_idx(j, q)
            if u < 2:
                start_idx(j + 2, (q + 2) % 4)
            scatter(q)

        plsc.subcore_barrier()
        pltpu.sync_copy(acc.at[pl.ds(row0, _RPT)],
                        out_hbm.at[c, pl.ds(row0, _RPT)])

    return k(dst_pad, zeros128, ones128)


def _agg_kernel(g, src_pad, dst_pad, zeros128, nch):
    epw = nch * _CH

    @functools.partial(
        pl.kernel,
        out_type=jax.ShapeDtypeStruct((_NSC, _NT, _D), jnp.float32),
        mesh=plsc.VectorSubcoreMesh(**_MESH_KW),
        scratch_types=[
            [pltpu.VMEM((_CH,), jnp.int32) for _ in range(4)],
            [pltpu.VMEM((_CH,), jnp.int32) for _ in range(4)],
            [pltpu.VMEM((_CH, _D), jnp.float32) for _ in range(2)],
            pltpu.VMEM_SHARED((_NT, _D), jnp.float32),
            [pltpu.SemaphoreType.DMA for _ in range(4)],
            [pltpu.SemaphoreType.DMA for _ in range(4)],
            [pltpu.SemaphoreType.DMA for _ in range(2)],
        ],
    )
    def k(g_hbm, src_hbm, dst_hbm, z_hbm, out_hbm,
          sidx, didx, rows, acc, ssem, dsem, gsem):
        c = lax.axis_index("c")
        s = lax.axis_index("s")
        row0 = s * _RPT
        pltpu.sync_copy(z_hbm.at[pl.ds(row0, _RPT)], acc.at[pl.ds(row0, _RPT)])
        plsc.subcore_barrier()
        base = (s * _NSC + c) * epw

        def start_idx(j, q):
            off = base + j * _CH
            pltpu.async_copy(src_hbm.at[pl.ds(off, _CH)], sidx[q], ssem[q])
            pltpu.async_copy(dst_hbm.at[pl.ds(off, _CH)], didx[q], dsem[q])

        def wait_idx(j, q):
            off = base + j * _CH
            pltpu.make_async_copy(src_hbm.at[pl.ds(off, _CH)],
                                  sidx[q], ssem[q]).wait()
            pltpu.make_async_copy(dst_hbm.at[pl.ds(off, _CH)],
                                  didx[q], dsem[q]).wait()

        def start_gather(q, b):
            pltpu.async_copy(g_hbm.at[sidx[q]], rows[b], gsem[b])

        def wait_gather(q, b):
            pltpu.make_async_copy(g_hbm.at[sidx[q]], rows[b], gsem[b]).wait()

        def scatter(q, b):
            pltpu.sync_copy(rows[b], acc.at[didx[q]], add=True)

        for q in range(4):
            start_idx(q, q)
        wait_idx(0, 0)
        start_gather(0, 0)
        wait_idx(1, 1)
        start_gather(1, 1)

        @pl.loop(0, (nch - 4) // 4)
        def _(jj):
            for u in range(4):
                q = u
                b = u % 2
                j4 = jj * 4 + u + 4
                wait_gather(q, b)
                scatter(q, b)
                start_idx(j4, q)
                wait_idx(jj * 4 + u + 2, (u + 2) % 4)
                start_gather((u + 2) % 4, b)

        for u in range(4):
            q = u
            b = u % 2
            wait_gather(q, b)
            scatter(q, b)
            if u < 2:
                wait_idx(nch - 2 + u, (u + 2) % 4)
                start_gather((u + 2) % 4, b)

        plsc.subcore_barrier()
        pltpu.sync_copy(acc.at[pl.ds(row0, _RPT)],
                        out_hbm.at[c, pl.ds(row0, _RPT)])

    return k(g, src_pad, dst_pad, zeros128)


def _mlp_body(x_ref, w1_ref, b1_ref, w2_ref, b2_ref, degp_ref, g0_ref, dc_ref):
    h = jnp.dot(x_ref[...], w1_ref[...], preferred_element_type=jnp.float32,
                    precision=lax.Precision.HIGHEST)
    h = jnp.maximum(h + b1_ref[...], 0.0)
    h = jnp.dot(h, w2_ref[...], preferred_element_type=jnp.float32,
                    precision=lax.Precision.HIGHEST)
    h = jnp.maximum(h + b2_ref[...], 0.0)
    deg = degp_ref[0, :, 0:1] + degp_ref[1, :, 0:1]
    dc = jnp.maximum(deg, 1.0)
    dc_ref[...] = dc
    g0_ref[...] = h * lax.rsqrt(dc)


def _mlp_call(x, W1, b1, W2, b2, degp):
    nb = _N // _BLK
    return pl.pallas_call(
        _mlp_body,
        grid=(nb,),
        in_specs=[
            pl.BlockSpec((_BLK, _D), lambda i: (i, 0)),
            pl.BlockSpec((_D, _D), lambda i: (0, 0)),
            pl.BlockSpec((1, _D), lambda i: (0, 0)),
            pl.BlockSpec((_D, _D), lambda i: (0, 0)),
            pl.BlockSpec((1, _D), lambda i: (0, 0)),
            pl.BlockSpec((_NSC, _BLK, _D), lambda i: (0, i, 0)),
        ],
        out_specs=[
            pl.BlockSpec((_BLK, _D), lambda i: (i, 0)),
            pl.BlockSpec((_BLK, 1), lambda i: (i, 0)),
        ],
        out_shape=[
            jax.ShapeDtypeStruct((_N, _D), jnp.float32),
            jax.ShapeDtypeStruct((_N, 1), jnp.float32),
        ],
    )(x, W1, b1.reshape(1, _D), W2, b2.reshape(1, _D), degp)


def _comb_body(g_ref, ap_ref, dc_ref, o_ref):
    agg = ap_ref[0] + ap_ref[1]
    o_ref[...] = g_ref[...] - agg / dc_ref[...]


def _comb_call(g, aggp, dc):
    nb = _N // _BLK
    return pl.pallas_call(
        _comb_body,
        grid=(nb,),
        in_specs=[
            pl.BlockSpec((_BLK, _D), lambda i: (i, 0)),
            pl.BlockSpec((_NSC, _BLK, _D), lambda i: (0, i, 0)),
            pl.BlockSpec((_BLK, 1), lambda i: (i, 0)),
        ],
        out_specs=pl.BlockSpec((_BLK, _D), lambda i: (i, 0)),
        out_shape=jax.ShapeDtypeStruct((_N, _D), jnp.float32),
    )(g, aggp, dc)


def _final_body(g0_ref, gs_ref, dc_ref, wm1_ref, bm1_ref,
                wm2_ref, bm2_ref, o_ref):
    dc = dc_ref[...]
    sdeg = jnp.sqrt(dc)
    wm1 = wm1_ref[...]
    wa = wm1[0:_D]
    wb = wm1[_D:2 * _D]
    wc = wm1[2 * _D:3 * _D]
    w0 = wa + _T1[0] * wb + _T2[0] * wc
    u = jnp.dot(sdeg * g0_ref[...], w0, preferred_element_type=jnp.float32,
                    precision=lax.Precision.HIGHEST)
    u = u + jnp.dot(sdeg * gs_ref[0], _T1[1] * wb + _T2[1] * wc,
                    preferred_element_type=jnp.float32,
                    precision=lax.Precision.HIGHEST)
    u = u + jnp.dot(sdeg * gs_ref[1], _T1[2] * wb + _T2[2] * wc,
                    preferred_element_type=jnp.float32,
                    precision=lax.Precision.HIGHEST)
    u = u + jnp.dot(sdeg * gs_ref[2], _T1[3] * wb + _T2[3] * wc,
                    preferred_element_type=jnp.float32,
                    precision=lax.Precision.HIGHEST)
    r = jnp.maximum(u + bm1_ref[...], 0.0)
    o_ref[...] = jnp.dot(r, wm2_ref[...],
                         preferred_element_type=jnp.float32,
                    precision=lax.Precision.HIGHEST) + bm2_ref[...]


def _final_call(g0, gs, dc, Wm1, bm1, Wm2, bm2):
    nb = _N // _BLK
    nc = Wm2.shape[1]
    return pl.pallas_call(
        _final_body,
        grid=(nb,),
        in_specs=[
            pl.BlockSpec((_BLK, _D), lambda i: (i, 0)),
            pl.BlockSpec((3, _BLK, _D), lambda i: (0, i, 0)),
            pl.BlockSpec((_BLK, 1), lambda i: (i, 0)),
            pl.BlockSpec((3 * _D, _D), lambda i: (0, 0)),
            pl.BlockSpec((1, _D), lambda i: (0, 0)),
            pl.BlockSpec((_D, nc), lambda i: (0, 0)),
            pl.BlockSpec((1, nc), lambda i: (0, 0)),
        ],
        out_specs=pl.BlockSpec((_BLK, nc), lambda i: (i, 0)),
        out_shape=jax.ShapeDtypeStruct((_N, nc), jnp.float32),
    )(g0, gs, dc, Wm1, bm1.reshape(1, _D), Wm2,
      bm2.reshape(1, nc))


def kernel(x, edge_index, W1, b1, W2, b2, Wm1, bm1, Wm2, bm2):
    src = edge_index[0].astype(jnp.int32)
    dst = edge_index[1].astype(jnp.int32)
    e = src.shape[0]
    unit = _NW * _CH * 4
    ep = ((e + unit - 1) // unit) * unit
    nch = ep // (_NW * _CH)
    npad = ep - e
    pad_i = jnp.arange(npad, dtype=jnp.int32)
    src_pad = jnp.concatenate([src, (pad_i * 997) % _N])
    dst_pad = jnp.concatenate([dst, _N + (pad_i % _NTRASH)])

    zeros128 = jnp.zeros((_NT, _D), jnp.float32)
    ones128 = jnp.ones((_CH, _D), jnp.float32)

    degp = _deg_kernel(dst_pad, zeros128, ones128, nch)
    g0, dc = _mlp_call(x, W1, b1, W2, b2, degp)

    def step(g, _):
        a = _agg_kernel(g, src_pad, dst_pad, zeros128, nch)
        gn = _comb_call(g, a, dc)
        return gn, gn

    _, gs = lax.scan(step, g0, xs=None, length=3)
    return _final_call(g0, gs, dc, Wm1, bm1, Wm2, bm2)


# speedup vs baseline: 8.9236x; 1.0714x over previous
import functools
import math

import jax
import jax.numpy as jnp
import numpy as np
from numpy.polynomial.chebyshev import chebfit
from jax import lax
from jax.experimental import pallas as pl
from jax.experimental.pallas import tpu as pltpu
from jax.experimental.pallas import tpu_sc as plsc

_N = 10000
_D = 128
_NTRASH = 112
_NT = _N + _NTRASH
_NSC = 2
_NSUB = 16
_NW = _NSC * _NSUB
_RPT = _NT // _NSUB
_CH = 128
_BLK = 400


def _chi_thetas():
    thetas = []
    for d in (2, 4):
        n = 2 * d
        i = n / 2 - 1
        m = 1.0 / (i + 2)
        n = int(n)

        def y(t):
            return (1.0 / (2 ** (n / 2) * math.gamma(n / 2))
                    * (t / m) ** (n / 2 - 1) * np.exp(-(t / m) / 2))

        xs = np.linspace(0.0, 2.0, 8001)
        s = np.trapz(y(xs), xs)
        lambs = np.linspace(0, 2, 500)
        coef = chebfit(lambs, y(lambs) / s, 3)
        thetas.append([float(v) for v in coef[::-1]])
    return thetas


_T1, _T2 = _chi_thetas()

_MESH_KW = dict(core_axis_name="c", subcore_axis_name="s")


def _deg_kernel(dst_pad, zeros128, ones128, nch):
    epw = nch * _CH

    @functools.partial(
        pl.kernel,
        out_type=jax.ShapeDtypeStruct((_NSC, _NT, _D), jnp.float32),
        mesh=plsc.VectorSubcoreMesh(**_MESH_KW),
        scratch_types=[
            [pltpu.VMEM((_CH,), jnp.int32) for _ in range(4)],
            pltpu.VMEM((_CH, _D), jnp.float32),
            pltpu.VMEM_SHARED((_NT, _D), jnp.float32),
            [pltpu.SemaphoreType.DMA for _ in range(4)],
        ],
    )
    def k(dst_hbm, z_hbm, ones_hbm, out_hbm, didx, ones_v, acc, dsem):
        c = lax.axis_index("c")
        s = lax.axis_index("s")
        row0 = s * _RPT
        pltpu.sync_copy(z_hbm.at[pl.ds(row0, _RPT)], acc.at[pl.ds(row0, _RPT)])
        pltpu.sync_copy(ones_hbm, ones_v)
        plsc.subcore_barrier()
        base = (s * _NSC + c) * epw

        def start_idx(j, q):
            pltpu.async_copy(dst_hbm.at[pl.ds(base + j * _CH, _CH)],
                             didx[q], dsem[q])

        def wait_idx(j, q):
            pltpu.make_async_copy(dst_hbm.at[pl.ds(base + j * _CH, _CH)],
                                  didx[q], dsem[q]).wait()

        def scatter(q):
            pltpu.sync_copy(ones_v, acc.at[didx[q]], add=True)

        start_idx(0, 0)
        start_idx(1, 1)

        @pl.loop(0, (nch - 4) // 4)
        def _(jj):
            for u in range(4):
                j = jj * 4 + u
                q = u % 4
                wait_idx(j, q)
                start_idx(j + 2, (u + 2) % 4)
                scatter(q)

        for u in range(4):
            j = nch - 4 + u
            q = (nch - 4 + u) % 4
            wait_idx(j, q)
            if u < 2:
                start_idx(j + 2, (q + 2) % 4)
            scatter(q)

        plsc.subcore_barrier()
        pltpu.sync_copy(acc.at[pl.ds(row0, _RPT)],
                        out_hbm.at[c, pl.ds(row0, _RPT)])

    return k(dst_pad, zeros128, ones128)


def _agg_kernel(g, src_pad, dst_pad, zeros128, nch):
    epw = nch * _CH

    @functools.partial(
        pl.kernel,
        out_type=jax.ShapeDtypeStruct((_NSC, _NT, _D), jnp.float32),
        mesh=plsc.VectorSubcoreMesh(**_MESH_KW),
        scratch_types=[
            [pltpu.VMEM((_CH,), jnp.int32) for _ in range(4)],
            [pltpu.VMEM((_CH,), jnp.int32) for _ in range(4)],
            [pltpu.VMEM((_CH, _D), jnp.float32) for _ in range(2)],
            pltpu.VMEM_SHARED((_NT, _D), jnp.float32),
            [pltpu.SemaphoreType.DMA for _ in range(4)],
            [pltpu.SemaphoreType.DMA for _ in range(4)],
            [pltpu.SemaphoreType.DMA for _ in range(2)],
        ],
    )
    def k(g_hbm, src_hbm, dst_hbm, z_hbm, out_hbm,
          sidx, didx, rows, acc, ssem, dsem, gsem):
        c = lax.axis_index("c")
        s = lax.axis_index("s")
        row0 = s * _RPT
        pltpu.sync_copy(z_hbm.at[pl.ds(row0, _RPT)], acc.at[pl.ds(row0, _RPT)])
        plsc.subcore_barrier()
        base = (s * _NSC + c) * epw

        def start_idx(j, q):
            off = base + j * _CH
            pltpu.async_copy(src_hbm.at[pl.ds(off, _CH)], sidx[q], ssem[q])
            pltpu.async_copy(dst_hbm.at[pl.ds(off, _CH)], didx[q], dsem[q])

        def wait_idx(j, q):
            off = base + j * _CH
            pltpu.make_async_copy(src_hbm.at[pl.ds(off, _CH)],
                                  sidx[q], ssem[q]).wait()
            pltpu.make_async_copy(dst_hbm.at[pl.ds(off, _CH)],
                                  didx[q], dsem[q]).wait()

        def start_gather(q, b):
            pltpu.async_copy(g_hbm.at[sidx[q]], rows[b], gsem[b])

        def wait_gather(q, b):
            pltpu.make_async_copy(g_hbm.at[sidx[q]], rows[b], gsem[b]).wait()

        def scatter(q, b):
            pltpu.sync_copy(rows[b], acc.at[didx[q]], add=True)

        for q in range(4):
            start_idx(q, q)
        wait_idx(0, 0)
        start_gather(0, 0)
        wait_idx(1, 1)
        start_gather(1, 1)

        @pl.loop(0, (nch - 4) // 4)
        def _(jj):
            for u in range(4):
                q = u
                b = u % 2
                j4 = jj * 4 + u + 4
                wait_gather(q, b)
                scatter(q, b)
                start_idx(j4, q)
                wait_idx(jj * 4 + u + 2, (u + 2) % 4)
                start_gather((u + 2) % 4, b)

        for u in range(4):
            q = u
            b = u % 2
            wait_gather(q, b)
            scatter(q, b)
            if u < 2:
                wait_idx(nch - 2 + u, (u + 2) % 4)
                start_gather((u + 2) % 4, b)

        plsc.subcore_barrier()
        pltpu.sync_copy(acc.at[pl.ds(row0, _RPT)],
                        out_hbm.at[c, pl.ds(row0, _RPT)])

    return k(g, src_pad, dst_pad, zeros128)


def _mlp_body(x_ref, w1_ref, b1_ref, w2_ref, b2_ref, degp_ref, g0_ref, dc_ref):
    h = jnp.dot(x_ref[...], w1_ref[...], preferred_element_type=jnp.float32,
                    precision=lax.Precision.HIGHEST)
    h = jnp.maximum(h + b1_ref[...], 0.0)
    h = jnp.dot(h, w2_ref[...], preferred_element_type=jnp.float32,
                    precision=lax.Precision.HIGHEST)
    h = jnp.maximum(h + b2_ref[...], 0.0)
    deg = degp_ref[0, :, 0:1] + degp_ref[1, :, 0:1]
    dc = jnp.maximum(deg, 1.0)
    dc_ref[...] = dc
    g0_ref[...] = h * lax.rsqrt(dc)


def _mlp_call(x, W1, b1, W2, b2, degp):
    nb = _N // _BLK
    return pl.pallas_call(
        _mlp_body,
        grid=(nb,),
        in_specs=[
            pl.BlockSpec((_BLK, _D), lambda i: (i, 0)),
            pl.BlockSpec((_D, _D), lambda i: (0, 0)),
            pl.BlockSpec((1, _D), lambda i: (0, 0)),
            pl.BlockSpec((_D, _D), lambda i: (0, 0)),
            pl.BlockSpec((1, _D), lambda i: (0, 0)),
            pl.BlockSpec((_NSC, _BLK, _D), lambda i: (0, i, 0)),
        ],
        out_specs=[
            pl.BlockSpec((_BLK, _D), lambda i: (i, 0)),
            pl.BlockSpec((_BLK, 1), lambda i: (i, 0)),
        ],
        out_shape=[
            jax.ShapeDtypeStruct((_N, _D), jnp.float32),
            jax.ShapeDtypeStruct((_N, 1), jnp.float32),
        ],
    )(x, W1, b1.reshape(1, _D), W2, b2.reshape(1, _D), degp)


def _comb_body(g_ref, ap_ref, dc_ref, o_ref):
    agg = ap_ref[0] + ap_ref[1]
    o_ref[...] = g_ref[...] - agg / dc_ref[...]


def _comb_call(g, aggp, dc):
    nb = _N // _BLK
    return pl.pallas_call(
        _comb_body,
        grid=(nb,),
        in_specs=[
            pl.BlockSpec((_BLK, _D), lambda i: (i, 0)),
            pl.BlockSpec((_NSC, _BLK, _D), lambda i: (0, i, 0)),
            pl.BlockSpec((_BLK, 1), lambda i: (i, 0)),
        ],
        out_specs=pl.BlockSpec((_BLK, _D), lambda i: (i, 0)),
        out_shape=jax.ShapeDtypeStruct((_N, _D), jnp.float32),
    )(g, aggp, dc)


def _final_body(g0_ref, g1_ref, g2_ref, ap_ref, dc_ref, wm1_ref, bm1_ref,
                wm2_ref, bm2_ref, o_ref):
    dc = dc_ref[...]
    sdeg = jnp.sqrt(dc)
    g3 = g2_ref[...] - (ap_ref[0] + ap_ref[1]) / dc
    wm1 = wm1_ref[...]
    wa = wm1[0:_D]
    wb = wm1[_D:2 * _D]
    wc = wm1[2 * _D:3 * _D]
    w0 = wa + _T1[0] * wb + _T2[0] * wc
    u = jnp.dot(sdeg * g0_ref[...], w0, preferred_element_type=jnp.float32,
                    precision=lax.Precision.HIGHEST)
    u = u + jnp.dot(sdeg * g1_ref[...], _T1[1] * wb + _T2[1] * wc,
                    preferred_element_type=jnp.float32,
                    precision=lax.Precision.HIGHEST)
    u = u + jnp.dot(sdeg * g2_ref[...], _T1[2] * wb + _T2[2] * wc,
                    preferred_element_type=jnp.float32,
                    precision=lax.Precision.HIGHEST)
    u = u + jnp.dot(sdeg * g3, _T1[3] * wb + _T2[3] * wc,
                    preferred_element_type=jnp.float32,
                    precision=lax.Precision.HIGHEST)
    r = jnp.maximum(u + bm1_ref[...], 0.0)
    o_ref[...] = jnp.dot(r, wm2_ref[...],
                         preferred_element_type=jnp.float32,
                    precision=lax.Precision.HIGHEST) + bm2_ref[...]


def _final_call(g0, g1, g2, aggp, dc, Wm1, bm1, Wm2, bm2):
    nb = _N // _BLK
    nc = Wm2.shape[1]
    return pl.pallas_call(
        _final_body,
        grid=(nb,),
        in_specs=[
            pl.BlockSpec((_BLK, _D), lambda i: (i, 0)),
            pl.BlockSpec((_BLK, _D), lambda i: (i, 0)),
            pl.BlockSpec((_BLK, _D), lambda i: (i, 0)),
            pl.BlockSpec((_NSC, _BLK, _D), lambda i: (0, i, 0)),
            pl.BlockSpec((_BLK, 1), lambda i: (i, 0)),
            pl.BlockSpec((3 * _D, _D), lambda i: (0, 0)),
            pl.BlockSpec((1, _D), lambda i: (0, 0)),
            pl.BlockSpec((_D, nc), lambda i: (0, 0)),
            pl.BlockSpec((1, nc), lambda i: (0, 0)),
        ],
        out_specs=pl.BlockSpec((_BLK, nc), lambda i: (i, 0)),
        out_shape=jax.ShapeDtypeStruct((_N, nc), jnp.float32),
    )(g0, g1, g2, aggp, dc, Wm1, bm1.reshape(1, _D), Wm2,
      bm2.reshape(1, nc))


def kernel(x, edge_index, W1, b1, W2, b2, Wm1, bm1, Wm2, bm2):
    src = edge_index[0].astype(jnp.int32)
    dst = edge_index[1].astype(jnp.int32)
    e = src.shape[0]
    unit = _NW * _CH * 4
    ep = ((e + unit - 1) // unit) * unit
    nch = ep // (_NW * _CH)
    npad = ep - e
    pad_i = jnp.arange(npad, dtype=jnp.int32)
    src_pad = jnp.concatenate([src, (pad_i * 997) % _N])
    dst_pad = jnp.concatenate([dst, _N + (pad_i % _NTRASH)])

    zeros128 = jnp.zeros((_NT, _D), jnp.float32)
    ones128 = jnp.ones((_CH, _D), jnp.float32)

    degp = _deg_kernel(dst_pad, zeros128, ones128, nch)
    g0, dc = _mlp_call(x, W1, b1, W2, b2, degp)

    a1 = _agg_kernel(g0, src_pad, dst_pad, zeros128, nch)
    g1 = _comb_call(g0, a1, dc)
    a2 = _agg_kernel(g1, src_pad, dst_pad, zeros128, nch)
    g2 = _comb_call(g1, a2, dc)
    a3 = _agg_kernel(g2, src_pad, dst_pad, zeros128, nch)
    return _final_call(g0, g1, g2, a3, dc, Wm1, bm1, Wm2, bm2)


# speedup vs baseline: 9.2264x; 1.0339x over previous
import functools
import math

import jax
import jax.numpy as jnp
import numpy as np
from numpy.polynomial.chebyshev import chebfit
from jax import lax
from jax.experimental import pallas as pl
from jax.experimental.pallas import tpu as pltpu
from jax.experimental.pallas import tpu_sc as plsc

_N = 10000
_D = 128
_NTRASH = 112
_NT = _N + _NTRASH
_NSC = 2
_NSUB = 16
_NW = _NSC * _NSUB
_RPT = _NT // _NSUB
_CH = 120
_BLK = 400


def _chi_thetas():
    thetas = []
    for d in (2, 4):
        n = 2 * d
        i = n / 2 - 1
        m = 1.0 / (i + 2)
        n = int(n)

        def y(t):
            return (1.0 / (2 ** (n / 2) * math.gamma(n / 2))
                    * (t / m) ** (n / 2 - 1) * np.exp(-(t / m) / 2))

        xs = np.linspace(0.0, 2.0, 8001)
        s = np.trapz(y(xs), xs)
        lambs = np.linspace(0, 2, 500)
        coef = chebfit(lambs, y(lambs) / s, 3)
        thetas.append([float(v) for v in coef[::-1]])
    return thetas


_T1, _T2 = _chi_thetas()

_MESH_KW = dict(core_axis_name="c", subcore_axis_name="s")


def _deg_kernel(dst_pad, zeros128, ones128, nch):
    epw = nch * _CH

    @functools.partial(
        pl.kernel,
        out_type=jax.ShapeDtypeStruct((_NSC, _NT, _D), jnp.float32),
        mesh=plsc.VectorSubcoreMesh(**_MESH_KW),
        scratch_types=[
            [pltpu.VMEM((_CH,), jnp.int32) for _ in range(4)],
            pltpu.VMEM((_CH, _D), jnp.float32),
            pltpu.VMEM_SHARED((_NT, _D), jnp.float32),
            [pltpu.SemaphoreType.DMA for _ in range(4)],
        ],
    )
    def k(dst_hbm, z_hbm, ones_hbm, out_hbm, didx, ones_v, acc, dsem):
        c = lax.axis_index("c")
        s = lax.axis_index("s")
        row0 = s * _RPT
        pltpu.sync_copy(z_hbm.at[pl.ds(row0, _RPT)], acc.at[pl.ds(row0, _RPT)])
        pltpu.sync_copy(ones_hbm, ones_v)
        plsc.subcore_barrier()
        base = (s * _NSC + c) * epw

        def start_idx(j, q):
            pltpu.async_copy(dst_hbm.at[pl.ds(base + j * _CH, _CH)],
                             didx[q], dsem[q])

        def wait_idx(j, q):
            pltpu.make_async_copy(dst_hbm.at[pl.ds(base + j * _CH, _CH)],
                                  didx[q], dsem[q]).wait()

        def scatter(q):
            pltpu.sync_copy(ones_v, acc.at[didx[q]], add=True)

        start_idx(0, 0)
        start_idx(1, 1)

        @pl.loop(0, (nch - 4) // 4)
        def _(jj):
            for u in range(4):
                j = jj * 4 + u
                q = u % 4
                wait_idx(j, q)
                start_idx(j + 2, (u + 2) % 4)
                scatter(q)

        for u in range(4):
            j = nch - 4 + u
            q = (nch - 4 + u) % 4
            wait_idx(j, q)
            if u < 2:
                start_idx(j + 2, (q + 2) % 4)
            scatter(q)

        plsc.subcore_barrier()
        pltpu.sync_copy(acc.at[pl.ds(row0, _RPT)],
                        out_hbm.at[c, pl.ds(row0, _RPT)])

    return k(dst_pad, zeros128, ones128)


def _agg_kernel(g, src_pad, dst_pad, zeros128, nch):
    epw = nch * _CH

    assert nch >= 12 and (nch - 6) % 6 == 0

    @functools.partial(
        pl.kernel,
        out_type=jax.ShapeDtypeStruct((_NSC, _NT, _D), jnp.float32),
        mesh=plsc.VectorSubcoreMesh(**_MESH_KW),
        scratch_types=[
            [pltpu.VMEM((_CH,), jnp.int32) for _ in range(6)],
            [pltpu.VMEM((_CH,), jnp.int32) for _ in range(6)],
            [pltpu.VMEM((_CH, _D), jnp.float32) for _ in range(3)],
            pltpu.VMEM_SHARED((_NT, _D), jnp.float32),
            [pltpu.SemaphoreType.DMA for _ in range(6)],
            [pltpu.SemaphoreType.DMA for _ in range(6)],
            [pltpu.SemaphoreType.DMA for _ in range(3)],
            [pltpu.SemaphoreType.DMA for _ in range(3)],
        ],
    )
    def k(g_hbm, src_hbm, dst_hbm, z_hbm, out_hbm,
          sidx, didx, rows, acc, isem_s, isem_d, gsem, ssem):
        c = lax.axis_index("c")
        s = lax.axis_index("s")
        row0 = s * _RPT
        pltpu.sync_copy(z_hbm.at[pl.ds(row0, _RPT)], acc.at[pl.ds(row0, _RPT)])
        plsc.subcore_barrier()
        base = (s * _NSC + c) * epw

        def start_idx(j, joff=0):
            q = j % 6
            off = base + (j + joff) * _CH
            pltpu.async_copy(src_hbm.at[pl.ds(off, _CH)], sidx[q], isem_s[q])
            pltpu.async_copy(dst_hbm.at[pl.ds(off, _CH)], didx[q], isem_d[q])

        def wait_idx(j, joff=0):
            q = j % 6
            off = base + (j + joff) * _CH
            pltpu.make_async_copy(src_hbm.at[pl.ds(off, _CH)],
                                  sidx[q], isem_s[q]).wait()
            pltpu.make_async_copy(dst_hbm.at[pl.ds(off, _CH)],
                                  didx[q], isem_d[q]).wait()

        def start_gather(j):
            pltpu.async_copy(g_hbm.at[sidx[j % 6]], rows[j % 3],
                             gsem[j % 3])

        def wait_gather(j):
            pltpu.make_async_copy(g_hbm.at[sidx[j % 6]], rows[j % 3],
                                  gsem[j % 3]).wait()

        def start_scatter(j):
            pltpu.async_copy(rows[j % 3], acc.at[didx[j % 6]],
                             ssem[j % 3], add=True)

        def wait_scatter(j):
            pltpu.make_async_copy(rows[j % 3], acc.at[didx[j % 6]],
                                  ssem[j % 3]).wait()

        for j in range(4):
            start_idx(j)
        wait_idx(0)
        start_gather(0)
        wait_idx(1)
        start_gather(1)
        wait_gather(0)
        start_scatter(0)
        wait_idx(2)
        start_gather(2)
        start_idx(4)
        wait_gather(1)
        start_scatter(1)
        wait_scatter(0)
        wait_idx(3)
        start_gather(3)
        start_idx(5)

        @pl.loop(0, (nch - 6) // 6)
        def _(it):
            joff = it * 6
            for u in range(6):
                j = u + 2
                wait_gather(j)
                start_scatter(j)
                wait_scatter(j - 1)
                wait_idx(j + 2, joff)
                start_gather(j + 2)
                start_idx(j + 4, joff)

        for j in range(nch - 4, nch):
            wait_gather(j)
            start_scatter(j)
            wait_scatter(j - 1)
            if j < nch - 2:
                wait_idx(j + 2)
                start_gather(j + 2)
        wait_scatter(nch - 1)

        plsc.subcore_barrier()
        pltpu.sync_copy(acc.at[pl.ds(row0, _RPT)],
                        out_hbm.at[c, pl.ds(row0, _RPT)])

    return k(g, src_pad, dst_pad, zeros128)


def _mlp_body(x_ref, w1_ref, b1_ref, w2_ref, b2_ref, degp_ref, g0_ref, dc_ref):
    h = jnp.dot(x_ref[...], w1_ref[...], preferred_element_type=jnp.float32,
                    precision=lax.Precision.HIGHEST)
    h = jnp.maximum(h + b1_ref[...], 0.0)
    h = jnp.dot(h, w2_ref[...], preferred_element_type=jnp.float32,
                    precision=lax.Precision.HIGHEST)
    h = jnp.maximum(h + b2_ref[...], 0.0)
    deg = degp_ref[0, :, 0:1] + degp_ref[1, :, 0:1]
    dc = jnp.maximum(deg, 1.0)
    dc_ref[...] = dc
    g0_ref[...] = h * lax.rsqrt(dc)


def _mlp_call(x, W1, b1, W2, b2, degp):
    nb = _N // _BLK
    return pl.pallas_call(
        _mlp_body,
        grid=(nb,),
        in_specs=[
            pl.BlockSpec((_BLK, _D), lambda i: (i, 0)),
            pl.BlockSpec((_D, _D), lambda i: (0, 0)),
            pl.BlockSpec((1, _D), lambda i: (0, 0)),
            pl.BlockSpec((_D, _D), lambda i: (0, 0)),
            pl.BlockSpec((1, _D), lambda i: (0, 0)),
            pl.BlockSpec((_NSC, _BLK, _D), lambda i: (0, i, 0)),
        ],
        out_specs=[
            pl.BlockSpec((_BLK, _D), lambda i: (i, 0)),
            pl.BlockSpec((_BLK, 1), lambda i: (i, 0)),
        ],
        out_shape=[
            jax.ShapeDtypeStruct((_N, _D), jnp.float32),
            jax.ShapeDtypeStruct((_N, 1), jnp.float32),
        ],
    )(x, W1, b1.reshape(1, _D), W2, b2.reshape(1, _D), degp)


def _comb_body(g_ref, ap_ref, dc_ref, o_ref):
    agg = ap_ref[0] + ap_ref[1]
    o_ref[...] = g_ref[...] - agg / dc_ref[...]


def _comb_call(g, aggp, dc):
    nb = _N // _BLK
    return pl.pallas_call(
        _comb_body,
        grid=(nb,),
        in_specs=[
            pl.BlockSpec((_BLK, _D), lambda i: (i, 0)),
            pl.BlockSpec((_NSC, _BLK, _D), lambda i: (0, i, 0)),
            pl.BlockSpec((_BLK, 1), lambda i: (i, 0)),
        ],
        out_specs=pl.BlockSpec((_BLK, _D), lambda i: (i, 0)),
        out_shape=jax.ShapeDtypeStruct((_N, _D), jnp.float32),
    )(g, aggp, dc)


def _final_body(g0_ref, g1_ref, g2_ref, ap_ref, dc_ref, wm1_ref, bm1_ref,
                wm2_ref, bm2_ref, o_ref):
    dc = dc_ref[...]
    sdeg = jnp.sqrt(dc)
    g3 = g2_ref[...] - (ap_ref[0] + ap_ref[1]) / dc
    wm1 = wm1_ref[...]
    wa = wm1[0:_D]
    wb = wm1[_D:2 * _D]
    wc = wm1[2 * _D:3 * _D]
    w0 = wa + _T1[0] * wb + _T2[0] * wc
    u = jnp.dot(sdeg * g0_ref[...], w0, preferred_element_type=jnp.float32,
                    precision=lax.Precision.HIGHEST)
    u = u + jnp.dot(sdeg * g1_ref[...], _T1[1] * wb + _T2[1] * wc,
                    preferred_element_type=jnp.float32,
                    precision=lax.Precision.HIGHEST)
    u = u + jnp.dot(sdeg * g2_ref[...], _T1[2] * wb + _T2[2] * wc,
                    preferred_element_type=jnp.float32,
                    precision=lax.Precision.HIGHEST)
    u = u + jnp.dot(sdeg * g3, _T1[3] * wb + _T2[3] * wc,
                    preferred_element_type=jnp.float32,
                    precision=lax.Precision.HIGHEST)
    r = jnp.maximum(u + bm1_ref[...], 0.0)
    o_ref[...] = jnp.dot(r, wm2_ref[...],
                         preferred_element_type=jnp.float32,
                    precision=lax.Precision.HIGHEST) + bm2_ref[...]


def _final_call(g0, g1, g2, aggp, dc, Wm1, bm1, Wm2, bm2):
    nb = _N // _BLK
    nc = Wm2.shape[1]
    return pl.pallas_call(
        _final_body,
        grid=(nb,),
        in_specs=[
            pl.BlockSpec((_BLK, _D), lambda i: (i, 0)),
            pl.BlockSpec((_BLK, _D), lambda i: (i, 0)),
            pl.BlockSpec((_BLK, _D), lambda i: (i, 0)),
            pl.BlockSpec((_NSC, _BLK, _D), lambda i: (0, i, 0)),
            pl.BlockSpec((_BLK, 1), lambda i: (i, 0)),
            pl.BlockSpec((3 * _D, _D), lambda i: (0, 0)),
            pl.BlockSpec((1, _D), lambda i: (0, 0)),
            pl.BlockSpec((_D, nc), lambda i: (0, 0)),
            pl.BlockSpec((1, nc), lambda i: (0, 0)),
        ],
        out_specs=pl.BlockSpec((_BLK, nc), lambda i: (i, 0)),
        out_shape=jax.ShapeDtypeStruct((_N, nc), jnp.float32),
    )(g0, g1, g2, aggp, dc, Wm1, bm1.reshape(1, _D), Wm2,
      bm2.reshape(1, nc))


def kernel(x, edge_index, W1, b1, W2, b2, Wm1, bm1, Wm2, bm2):
    src = edge_index[0].astype(jnp.int32)
    dst = edge_index[1].astype(jnp.int32)
    e = src.shape[0]
    nch = -(-e // (_NW * _CH))
    nch = -(-nch // 12) * 12
    ep = nch * _NW * _CH
    npad = ep - e
    pad_i = jnp.arange(npad, dtype=jnp.int32)
    src_pad = jnp.concatenate([src, (pad_i * 997) % _N])
    dst_pad = jnp.concatenate([dst, _N + (pad_i % _NTRASH)])

    zeros128 = jnp.zeros((_NT, _D), jnp.float32)
    ones128 = jnp.ones((_CH, _D), jnp.float32)

    degp = _deg_kernel(dst_pad, zeros128, ones128, nch)
    g0, dc = _mlp_call(x, W1, b1, W2, b2, degp)

    a1 = _agg_kernel(g0, src_pad, dst_pad, zeros128, nch)
    g1 = _comb_call(g0, a1, dc)
    a2 = _agg_kernel(g1, src_pad, dst_pad, zeros128, nch)
    g2 = _comb_call(g1, a2, dc)
    a3 = _agg_kernel(g2, src_pad, dst_pad, zeros128, nch)
    return _final_call(g0, g1, g2, a3, dc, Wm1, bm1, Wm2, bm2)


# speedup vs baseline: 10.3312x; 1.1197x over previous
import dataclasses
import functools
import math

import jax
import jax.numpy as jnp
import numpy as np
from numpy.polynomial.chebyshev import chebfit
from jax import lax
from jax.experimental import pallas as pl
from jax.experimental.pallas import tpu as pltpu
from jax.experimental.pallas import tpu_sc as plsc

_N = 10000
_D = 128
_NTRASH = 240
_NT = _N + _NTRASH
_NSC = 2
_NSUB = 16
_NW = _NSC * _NSUB
_RPT = _NT // _NSUB
_CH = 120
_BLK = 400


def _chi_thetas():
    thetas = []
    for d in (2, 4):
        n = 2 * d
        i = n / 2 - 1
        m = 1.0 / (i + 2)
        n = int(n)

        def y(t):
            return (1.0 / (2 ** (n / 2) * math.gamma(n / 2))
                    * (t / m) ** (n / 2 - 1) * np.exp(-(t / m) / 2))

        xs = np.linspace(0.0, 2.0, 8001)
        s = np.trapz(y(xs), xs)
        lambs = np.linspace(0, 2, 500)
        coef = chebfit(lambs, y(lambs) / s, 3)
        thetas.append([float(v) for v in coef[::-1]])
    return thetas


_T1, _T2 = _chi_thetas()

_MESH_KW = dict(core_axis_name="c", subcore_axis_name="s")


def _deg_kernel(dst_pad, zeros80, nch):
    epw = nch * _CH
    nvec = epw // 16
    nhr = _NT // _D
    rpt2 = nhr // _NSUB

    cp = pltpu.CompilerParams()
    if "needs_layout_passes" in pltpu.CompilerParams.__dataclass_fields__:
        cp = dataclasses.replace(cp, needs_layout_passes=False)

    @functools.partial(
        pl.kernel,
        out_type=jax.ShapeDtypeStruct((_NSC, _NT, _D), jnp.float32),
        compiler_params=cp,
        mesh=plsc.VectorSubcoreMesh(**_MESH_KW),
        scratch_types=[
            pltpu.VMEM((epw,), jnp.int32),
            pltpu.VMEM((nhr, _D), jnp.float32),
            pltpu.VMEM((nhr,), jnp.int32),
            pltpu.VMEM((rpt2, _D), jnp.float32),
            pltpu.VMEM((_RPT, _D), jnp.float32),
            pltpu.VMEM_SHARED((nhr, _D), jnp.float32),
        ],
    )
    def k(dst_hbm, z_hbm, out_hbm, dall, hist, rowidx, degv, stage, acc):
        c = lax.axis_index("c")
        s = lax.axis_index("s")
        @pl.when(s < nhr // 8)
        def _():
            pltpu.sync_copy(z_hbm.at[pl.ds(s * 8, 8)],
                            acc.at[pl.ds(s * 8, 8)])

        base = (s * _NSC + c) * epw
        pltpu.sync_copy(dst_hbm.at[pl.ds(base, epw)], dall)

        zeros = jnp.zeros((16,), jnp.float32)
        iota = lax.iota(jnp.int32, 16)

        for r in range(nhr // 16):
            rowidx[pl.ds(r * 16, 16)] = iota + r * 16

        @pl.loop(0, nhr)
        def _(r):
            for kk in range(_D // 16):
                hist[r, pl.ds(kk * 16, 16)] = zeros

        ones = jnp.ones((16,), jnp.float32)

        @pl.loop(0, nvec)
        def _(v):
            iv = dall[pl.ds(v * 16, 16)]
            rv = lax.shift_right_logical(iv, 7)
            cv = lax.bitwise_and(iv, 127)
            plsc.addupdate_scatter(hist, [rv, cv], ones)

        plsc.subcore_barrier()
        pltpu.sync_copy(hist, acc.at[rowidx], add=True)
        plsc.subcore_barrier()

        pltpu.sync_copy(acc.at[pl.ds(s * rpt2, rpt2)], degv)

        @pl.loop(0, rpt2)
        def _(rr):
            @pl.loop(0, _D // 16)
            def _(cb):
                v = degv[rr, pl.ds(cb * 16, 16)]
                for l in range(16):
                    bc = jnp.full((16,), v[l], jnp.float32)
                    row = rr * _D + cb * 16 + l
                    for kk in range(_D // 16):
                        stage[row, pl.ds(kk * 16, 16)] = bc

        pltpu.sync_copy(stage, out_hbm.at[c, pl.ds(s * _RPT, _RPT)])

    return k(dst_pad, zeros80)


def _agg_kernel(g, src_pad, dst_pad, zeros128, nch):
    epw = nch * _CH

    assert nch >= 12 and (nch - 6) % 6 == 0

    @functools.partial(
        pl.kernel,
        out_type=jax.ShapeDtypeStruct((_NSC, _NT, _D), jnp.float32),
        mesh=plsc.VectorSubcoreMesh(**_MESH_KW),
        scratch_types=[
            [pltpu.VMEM((_CH,), jnp.int32) for _ in range(6)],
            [pltpu.VMEM((_CH,), jnp.int32) for _ in range(6)],
            [pltpu.VMEM((_CH, _D), jnp.float32) for _ in range(3)],
            pltpu.VMEM_SHARED((_NT, _D), jnp.float32),
            [pltpu.SemaphoreType.DMA for _ in range(6)],
            [pltpu.SemaphoreType.DMA for _ in range(6)],
            [pltpu.SemaphoreType.DMA for _ in range(3)],
            [pltpu.SemaphoreType.DMA for _ in range(3)],
        ],
    )
    def k(g_hbm, src_hbm, dst_hbm, z_hbm, out_hbm,
          sidx, didx, rows, acc, isem_s, isem_d, gsem, ssem):
        c = lax.axis_index("c")
        s = lax.axis_index("s")
        row0 = s * _RPT
        pltpu.sync_copy(z_hbm.at[pl.ds(row0, _RPT)], acc.at[pl.ds(row0, _RPT)])
        plsc.subcore_barrier()
        base = (s * _NSC + c) * epw

        def start_idx(j, joff=0):
            q = j % 6
            off = base + (j + joff) * _CH
            pltpu.async_copy(src_hbm.at[pl.ds(off, _CH)], sidx[q], isem_s[q])
            pltpu.async_copy(dst_hbm.at[pl.ds(off, _CH)], didx[q], isem_d[q])

        def wait_idx(j, joff=0):
            q = j % 6
            off = base + (j + joff) * _CH
            pltpu.make_async_copy(src_hbm.at[pl.ds(off, _CH)],
                                  sidx[q], isem_s[q]).wait()
            pltpu.make_async_copy(dst_hbm.at[pl.ds(off, _CH)],
                                  didx[q], isem_d[q]).wait()

        def start_gather(j):
            pltpu.async_copy(g_hbm.at[sidx[j % 6]], rows[j % 3],
                             gsem[j % 3])

        def wait_gather(j):
            pltpu.make_async_copy(g_hbm.at[sidx[j % 6]], rows[j % 3],
                                  gsem[j % 3]).wait()

        def start_scatter(j):
            pltpu.async_copy(rows[j % 3], acc.at[didx[j % 6]],
                             ssem[j % 3], add=True)

        def wait_scatter(j):
            pltpu.make_async_copy(rows[j % 3], acc.at[didx[j % 6]],
                                  ssem[j % 3]).wait()

        for j in range(4):
            start_idx(j)
        wait_idx(0)
        start_gather(0)
        wait_idx(1)
        start_gather(1)
        wait_gather(0)
        start_scatter(0)
        wait_idx(2)
        start_gather(2)
        start_idx(4)
        wait_gather(1)
        start_scatter(1)
        wait_scatter(0)
        wait_idx(3)
        start_gather(3)
        start_idx(5)

        @pl.loop(0, (nch - 6) // 6)
        def _(it):
            joff = it * 6
            for u in range(6):
                j = u + 2
                wait_gather(j)
                start_scatter(j)
                wait_scatter(j - 1)
                wait_idx(j + 2, joff)
                start_gather(j + 2)
                start_idx(j + 4, joff)

        for j in range(nch - 4, nch):
            wait_gather(j)
            start_scatter(j)
            wait_scatter(j - 1)
            if j < nch - 2:
                wait_idx(j + 2)
                start_gather(j + 2)
        wait_scatter(nch - 1)

        plsc.subcore_barrier()
        pltpu.sync_copy(acc.at[pl.ds(row0, _RPT)],
                        out_hbm.at[c, pl.ds(row0, _RPT)])

    return k(g, src_pad, dst_pad, zeros128)


def _mlp_body(x_ref, w1_ref, b1_ref, w2_ref, b2_ref, degp_ref, g0_ref, dc_ref):
    h = jnp.dot(x_ref[...], w1_ref[...], preferred_element_type=jnp.float32,
                    precision=lax.Precision.HIGHEST)
    h = jnp.maximum(h + b1_ref[...], 0.0)
    h = jnp.dot(h, w2_ref[...], preferred_element_type=jnp.float32,
                    precision=lax.Precision.HIGHEST)
    h = jnp.maximum(h + b2_ref[...], 0.0)
    deg = degp_ref[0, :, 0:1] + degp_ref[1, :, 0:1]
    dc = jnp.maximum(deg, 1.0)
    dc_ref[...] = dc
    g0_ref[...] = h * lax.rsqrt(dc)


def _mlp_call(x, W1, b1, W2, b2, degp):
    nb = _N // _BLK
    return pl.pallas_call(
        _mlp_body,
        grid=(nb,),
        in_specs=[
            pl.BlockSpec((_BLK, _D), lambda i: (i, 0)),
            pl.BlockSpec((_D, _D), lambda i: (0, 0)),
            pl.BlockSpec((1, _D), lambda i: (0, 0)),
            pl.BlockSpec((_D, _D), lambda i: (0, 0)),
            pl.BlockSpec((1, _D), lambda i: (0, 0)),
            pl.BlockSpec((_NSC, _BLK, _D), lambda i: (0, i, 0)),
        ],
        out_specs=[
            pl.BlockSpec((_BLK, _D), lambda i: (i, 0)),
            pl.BlockSpec((_BLK, 1), lambda i: (i, 0)),
        ],
        out_shape=[
            jax.ShapeDtypeStruct((_N, _D), jnp.float32),
            jax.ShapeDtypeStruct((_N, 1), jnp.float32),
        ],
    )(x, W1, b1.reshape(1, _D), W2, b2.reshape(1, _D), degp)


def _comb_body(g_ref, ap_ref, dc_ref, o_ref):
    agg = ap_ref[0] + ap_ref[1]
    o_ref[...] = g_ref[...] - agg / dc_ref[...]


def _comb_call(g, aggp, dc):
    nb = _N // _BLK
    return pl.pallas_call(
        _comb_body,
        grid=(nb,),
        in_specs=[
            pl.BlockSpec((_BLK, _D), lambda i: (i, 0)),
            pl.BlockSpec((_NSC, _BLK, _D), lambda i: (0, i, 0)),
            pl.BlockSpec((_BLK, 1), lambda i: (i, 0)),
        ],
        out_specs=pl.BlockSpec((_BLK, _D), lambda i: (i, 0)),
        out_shape=jax.ShapeDtypeStruct((_N, _D), jnp.float32),
    )(g, aggp, dc)


def _final_body(g0_ref, g1_ref, g2_ref, ap_ref, dc_ref, wm1_ref, bm1_ref,
                wm2_ref, bm2_ref, o_ref):
    dc = dc_ref[...]
    sdeg = jnp.sqrt(dc)
    g3 = g2_ref[...] - (ap_ref[0] + ap_ref[1]) / dc
    wm1 = wm1_ref[...]
    wa = wm1[0:_D]
    wb = wm1[_D:2 * _D]
    wc = wm1[2 * _D:3 * _D]
    w0 = wa + _T1[0] * wb + _T2[0] * wc
    u = jnp.dot(sdeg * g0_ref[...], w0, preferred_element_type=jnp.float32,
                    precision=lax.Precision.HIGHEST)
    u = u + jnp.dot(sdeg * g1_ref[...], _T1[1] * wb + _T2[1] * wc,
                    preferred_element_type=jnp.float32,
                    precision=lax.Precision.HIGHEST)
    u = u + jnp.dot(sdeg * g2_ref[...], _T1[2] * wb + _T2[2] * wc,
                    preferred_element_type=jnp.float32,
                    precision=lax.Precision.HIGHEST)
    u = u + jnp.dot(sdeg * g3, _T1[3] * wb + _T2[3] * wc,
                    preferred_element_type=jnp.float32,
                    precision=lax.Precision.HIGHEST)
    r = jnp.maximum(u + bm1_ref[...], 0.0)
    o_ref[...] = jnp.dot(r, wm2_ref[...],
                         preferred_element_type=jnp.float32,
                    precision=lax.Precision.HIGHEST) + bm2_ref[...]


def _final_call(g0, g1, g2, aggp, dc, Wm1, bm1, Wm2, bm2):
    nb = _N // _BLK
    nc = Wm2.shape[1]
    return pl.pallas_call(
        _final_body,
        grid=(nb,),
        in_specs=[
            pl.BlockSpec((_BLK, _D), lambda i: (i, 0)),
            pl.BlockSpec((_BLK, _D), lambda i: (i, 0)),
            pl.BlockSpec((_BLK, _D), lambda i: (i, 0)),
            pl.BlockSpec((_NSC, _BLK, _D), lambda i: (0, i, 0)),
            pl.BlockSpec((_BLK, 1), lambda i: (i, 0)),
            pl.BlockSpec((3 * _D, _D), lambda i: (0, 0)),
            pl.BlockSpec((1, _D), lambda i: (0, 0)),
            pl.BlockSpec((_D, nc), lambda i: (0, 0)),
            pl.BlockSpec((1, nc), lambda i: (0, 0)),
        ],
        out_specs=pl.BlockSpec((_BLK, nc), lambda i: (i, 0)),
        out_shape=jax.ShapeDtypeStruct((_N, nc), jnp.float32),
    )(g0, g1, g2, aggp, dc, Wm1, bm1.reshape(1, _D), Wm2,
      bm2.reshape(1, nc))


def kernel(x, edge_index, W1, b1, W2, b2, Wm1, bm1, Wm2, bm2):
    src = edge_index[0].astype(jnp.int32)
    dst = edge_index[1].astype(jnp.int32)
    e = src.shape[0]
    nch = -(-e // (_NW * _CH))
    nch = -(-nch // 12) * 12
    ep = nch * _NW * _CH
    npad = ep - e
    pad_i = jnp.arange(npad, dtype=jnp.int32)
    src_pad = jnp.concatenate([src, (pad_i * 997) % _N])
    dst_pad = jnp.concatenate([dst, _N + (pad_i % _NTRASH)])

    zeros128 = jnp.zeros((_NT, _D), jnp.float32)
    zeros80 = jnp.zeros((_NT // _D, _D), jnp.float32)

    degp = _deg_kernel(dst_pad, zeros80, nch)
    g0, dc = _mlp_call(x, W1, b1, W2, b2, degp)

    a1 = _agg_kernel(g0, src_pad, dst_pad, zeros128, nch)
    g1 = _comb_call(g0, a1, dc)
    a2 = _agg_kernel(g1, src_pad, dst_pad, zeros128, nch)
    g2 = _comb_call(g1, a2, dc)
    a3 = _agg_kernel(g2, src_pad, dst_pad, zeros128, nch)
    return _final_call(g0, g1, g2, a3, dc, Wm1, bm1, Wm2, bm2)
